# baseline (speedup 1.0000x reference)
import functools

import jax
import jax.numpy as jnp
from jax import lax
from jax.experimental import pallas as pl
from jax.experimental.pallas import tpu as pltpu
from jax.experimental.pallas import tpu_sc as plsc

B = 1024
DIM = 128
H1 = 256
H2 = 256
N = 10000
E = 160000
EPS = 1e-15

NC = 2
NS = 16
CHUNK = 128
EP = 163840
EC1 = EP // NS
CQ1 = EC1 // CHUNK
EC2 = EP // (NC * NS)
CQ2 = EC2 // CHUNK
NPT = 624

_INTERP = False


def _artanh(z):
    z = jnp.clip(z, -1.0 + 1e-7, 1.0 - 1e-7)
    return 0.5 * jnp.log((1.0 + z) / (1.0 - z))


def _rownorm(sq):
    return jnp.clip(jnp.sqrt(sq), EPS, None)


def _tc1_body(x_ref, w_ref, b_ref, asrc_ref, adst_ref, c_ref, h_ref, als_ref, ald_ref):
    x = x_ref[...]
    c = c_ref[0]
    sc = jnp.sqrt(c)
    n = _rownorm(jnp.sum(x * x, axis=1, keepdims=True))
    maxn = (1.0 - 1e-5) / sc
    p = jnp.where(n > maxn, x / n * maxn, x)
    pn = _rownorm(jnp.sum(p * p, axis=1, keepdims=True))
    xt = _artanh(sc * pn) * p / (sc * pn)
    h = jnp.dot(xt, w_ref[...], preferred_element_type=jnp.float32) + b_ref[...]
    h_ref[...] = h
    als_ref[...] = jnp.sum(h * asrc_ref[...], axis=1, keepdims=True)
    ald_ref[...] = jnp.sum(h * adst_ref[...], axis=1, keepdims=True)


def _tc1(x, W1, b1, a1s, a1d, c_in):
    RB = 400
    grid = (N // RB,)
    return pl.pallas_call(
        _tc1_body,
        grid=grid,
        in_specs=[
            pl.BlockSpec((RB, H1), lambda i: (i, 0)),
            pl.BlockSpec((H1, H2), lambda i: (0, 0)),
            pl.BlockSpec((1, H2), lambda i: (0, 0)),
            pl.BlockSpec((1, H2), lambda i: (0, 0)),
            pl.BlockSpec((1, H2), lambda i: (0, 0)),
            pl.BlockSpec(memory_space=pltpu.SMEM),
        ],
        out_specs=[
            pl.BlockSpec((RB, H2), lambda i: (i, 0)),
            pl.BlockSpec((RB, 1), lambda i: (i, 0)),
            pl.BlockSpec((RB, 1), lambda i: (i, 0)),
        ],
        out_shape=[
            jax.ShapeDtypeStruct((N, H2), jnp.float32),
            jax.ShapeDtypeStruct((N, 1), jnp.float32),
            jax.ShapeDtypeStruct((N, 1), jnp.float32),
        ],
        interpret=_INTERP,
    )(x, W1, b1.reshape(1, H2), a1s.reshape(1, H2), a1d.reshape(1, H2), c_in)


def _tc2_body(q0_ref, q1_ref, q2_ref, q3_ref, ws_ref, w2_ref, b2_ref,
              asrc_ref, adst_ref, cin_ref, cout_ref, h_ref, als_ref, ald_ref):
    w = ws_ref[...] + 1e-16
    qs = [jnp.maximum(q_ref[...] / w, 0.0)
          for q_ref in (q0_ref, q1_ref, q2_ref, q3_ref)]
    co = cout_ref[0]
    sco = jnp.sqrt(co)
    sq = sum(jnp.sum(q * q, axis=1, keepdims=True) for q in qs)
    un = _rownorm(sq)
    gs = jnp.tanh(sco * un) / (sco * un)
    gn = _rownorm(sq * gs * gs)
    maxn_o = (1.0 - 1e-5) / sco
    scale = gs * jnp.where(gn > maxn_o, maxn_o / gn, 1.0)
    gsq = sq * scale * scale
    ci = cin_ref[0]
    sci = jnp.sqrt(ci)
    n = _rownorm(gsq)
    maxn_i = (1.0 - 1e-5) / sci
    pscale = jnp.where(n > maxn_i, maxn_i / n, 1.0)
    pn = _rownorm(gsq * pscale * pscale)
    ls = scale * pscale * (_artanh(sci * pn) / (sci * pn))
    h = b2_ref[...]
    for qi, q in enumerate(qs):
        h = h + jnp.dot(q * ls, w2_ref[qi * FW:(qi + 1) * FW, :],
                        preferred_element_type=jnp.float32)
    h_ref[...] = h
    als_ref[...] = jnp.sum(h * asrc_ref[...], axis=1, keepdims=True)
    ald_ref[...] = jnp.sum(h * adst_ref[...], axis=1, keepdims=True)


def _tc2(agg_slabs, wsum, W2, b2, a2s, a2d, c_in, c_out):
    RB = 400
    grid = (N // RB,)
    nb = N // RB
    return pl.pallas_call(
        _tc2_body,
        grid=grid,
        in_specs=[
            pl.BlockSpec((RB, FW), lambda i: (i, 0)),
            pl.BlockSpec((RB, FW), lambda i: (i + nb, 0)),
            pl.BlockSpec((RB, FW), lambda i: (i + 2 * nb, 0)),
            pl.BlockSpec((RB, FW), lambda i: (i + 3 * nb, 0)),
            pl.BlockSpec((RB, 1), lambda i: (i, 0)),
            pl.BlockSpec((H2, DIM), lambda i: (0, 0)),
            pl.BlockSpec((1, DIM), lambda i: (0, 0)),
            pl.BlockSpec((1, DIM), lambda i: (0, 0)),
            pl.BlockSpec((1, DIM), lambda i: (0, 0)),
            pl.BlockSpec(memory_space=pltpu.SMEM),
            pl.BlockSpec(memory_space=pltpu.SMEM),
        ],
        out_specs=[
            pl.BlockSpec((RB, DIM), lambda i: (i, 0)),
            pl.BlockSpec((RB, 1), lambda i: (i, 0)),
            pl.BlockSpec((RB, 1), lambda i: (i, 0)),
        ],
        out_shape=[
            jax.ShapeDtypeStruct((N, DIM), jnp.float32),
            jax.ShapeDtypeStruct((N, 1), jnp.float32),
            jax.ShapeDtypeStruct((N, 1), jnp.float32),
        ],
        interpret=_INTERP,
    )(agg_slabs, agg_slabs, agg_slabs, agg_slabs, wsum.reshape(N, 1), W2,
      b2.reshape(1, DIM), a2s.reshape(1, DIM), a2d.reshape(1, DIM),
      c_in, c_out)


def _tc3a_body(p0_ref, p1_ref, w_ref, co_ref, h_ref):
    co = co_ref[0]
    sco = jnp.sqrt(co)
    w = w_ref[...] + 1e-16
    r0 = jnp.maximum(p0_ref[...] / w, 0.0)
    r1 = jnp.maximum(p1_ref[...] / w, 0.0)
    sq = (jnp.sum(r0 * r0, axis=1, keepdims=True)
          + jnp.sum(r1 * r1, axis=1, keepdims=True))
    un = _rownorm(sq)
    gs = jnp.tanh(sco * un) / (sco * un)
    gn = _rownorm(sq * gs * gs)
    maxn = (1.0 - 1e-5) / sco
    scale = gs * jnp.where(gn > maxn, maxn / gn, 1.0)
    h_ref[:, 0:FW] = r0 * scale
    h_ref[:, FW:DIM] = r1 * scale


def _tc3a(p_slabs, w, c_out):
    RB = 400
    nb = N // RB
    return pl.pallas_call(
        _tc3a_body,
        grid=(N // RB,),
        in_specs=[
            pl.BlockSpec((RB, FW), lambda k: (k, 0)),
            pl.BlockSpec((RB, FW), lambda k: (k + nb, 0)),
            pl.BlockSpec((RB, 1), lambda k: (k, 0)),
            pl.BlockSpec(memory_space=pltpu.SMEM),
        ],
        out_specs=pl.BlockSpec((RB, DIM), lambda k: (k, 0)),
        out_shape=jax.ShapeDtypeStruct((N, DIM), jnp.float32),
        interpret=_INTERP,
    )(p_slabs, p_slabs, w, c_out)


def _tc3b_body(i_ref, h_ref, u_ref, co_ref, r_ref, t_ref, o_ref):
    co = co_ref[0]
    sco = jnp.sqrt(co)
    maxn = (1.0 - 1e-5) / sco
    ih = jnp.dot(i_ref[...], h_ref[...], preferred_element_type=jnp.float32)
    inorm = _rownorm(jnp.sum(ih * ih, axis=1, keepdims=True))
    ih = jnp.where(inorm > maxn, ih / inorm * maxn, ih)
    u = u_ref[...]
    un2 = _rownorm(jnp.sum(u * u, axis=1, keepdims=True))
    uh = jnp.tanh(sco * un2) / (sco * un2) * u
    uhn = _rownorm(jnp.sum(uh * uh, axis=1, keepdims=True))
    uh = jnp.where(uhn > maxn, uh / uhn * maxn, uh)
    x = -uh
    x2 = jnp.sum(x * x, axis=1, keepdims=True)
    y2 = jnp.sum(ih * ih, axis=1, keepdims=True)
    xy = jnp.sum(x * ih, axis=1, keepdims=True)
    num = (1.0 + 2.0 * co * xy + co * y2) * x + (1.0 - co * x2) * ih
    den = 1.0 + 2.0 * co * xy + (co * co) * x2 * y2
    m = num / jnp.clip(den, EPS, None)
    d = jnp.sqrt(jnp.clip(jnp.sum(m * m, axis=1, keepdims=True), 0.0, None))
    dist = 2.0 / sco * _artanh(sco * d)
    o_ref[...] = 1.0 / (jnp.exp((dist - r_ref[0]) / t_ref[0]) + 1.0)


def _tc3b(i_mat, h2, u_emb, c_out, r, t):
    MB = 256
    return pl.pallas_call(
        _tc3b_body,
        grid=(B // MB,),
        in_specs=[
            pl.BlockSpec((MB, N), lambda m: (m, 0)),
            pl.BlockSpec((N, DIM), lambda m: (0, 0)),
            pl.BlockSpec((MB, DIM), lambda m: (m, 0)),
            pl.BlockSpec(memory_space=pltpu.SMEM),
            pl.BlockSpec(memory_space=pltpu.SMEM),
            pl.BlockSpec(memory_space=pltpu.SMEM),
        ],
        out_specs=pl.BlockSpec((MB, 1), lambda m: (m, 0)),
        out_shape=jax.ShapeDtypeStruct((B, 1), jnp.float32),
        interpret=_INTERP,
    )(i_mat, h2, u_emb, c_out, r, t)


def _mesh():
    return plsc.VectorSubcoreMesh(core_axis_name="c", subcore_axis_name="s",
                                  num_cores=NC, num_subcores=NS)

FW = 64


def _zero_fill(buf, nvec):
    zv = jnp.zeros((16,), jnp.float32)
    rows = buf.shape[-1] // 16 if len(buf.shape) == 2 else 0

    def zb(j, carry):
        if len(buf.shape) == 2:
            buf[j // rows, pl.ds((j % rows) * 16, 16)] = zv
        else:
            buf[pl.ds(j * 16, 16)] = zv
        return carry

    lax.fori_loop(0, nvec, zb, 0)


def _edge_weights(asg_v, adg_v, w_v, ebase, nvec, lanes):
    def wb(j, carry):
        o = j * 16
        av = asg_v[pl.ds(o, 16)] + adg_v[pl.ds(o, 16)]
        e = jnp.where(av > 0, av, 0.2 * av)
        w = jnp.exp(e)
        gid = ebase + o + lanes
        w_v[pl.ds(o, 16)] = jnp.where(gid < E, w, 0.0)
        return carry

    lax.fori_loop(0, nvec, wb, 0)


def _zero_agg(zrows_v, agg_sp, s):
    def zc(kk, carry):
        pltpu.sync_copy(zrows_v, agg_sp.at[pl.ds(s * NPT + kk * 16, 16)])
        return carry

    lax.fori_loop(0, NPT // 16, zc, 0)

    @pl.when(s == NS - 1)
    def _():
        pltpu.sync_copy(zrows_v, agg_sp.at[pl.ds(NS * NPT, 16)])


def _sc_edge(h_slabs, asrc, adst, src1d, dst1d, nslab):
    spc = nslab // NC

    @functools.partial(
        pl.kernel,
        out_type=[jax.ShapeDtypeStruct((nslab * N, FW), jnp.float32),
                  jax.ShapeDtypeStruct((NC, N), jnp.float32)],
        mesh=_mesh(),
        compiler_params=pltpu.CompilerParams(use_tc_tiling_on_sc=False),
        scratch_types=[
            pltpu.VMEM((EC1,), jnp.int32),
            pltpu.VMEM((EC1,), jnp.int32),
            pltpu.VMEM((CQ1, CHUNK), jnp.int32),
            pltpu.VMEM((EC1,), jnp.float32),
            pltpu.VMEM((EC1,), jnp.float32),
            pltpu.VMEM((EC1 + 16,), jnp.float32),
            pltpu.VMEM((CQ1, CHUNK), jnp.int32),
            pltpu.VMEM((CHUNK, FW), jnp.float32),
            pltpu.VMEM((16, FW), jnp.float32),
            pltpu.VMEM((2000,), jnp.float32),
            pltpu.VMEM_SHARED((N, FW), jnp.float32),
            pltpu.VMEM_SHARED((N,), jnp.float32),
            pltpu.SemaphoreType.DMA,
            pltpu.SemaphoreType.DMA,
        ],
    )
    def k(h_hbm, as_hbm, ad_hbm, src_hbm, dst_hbm, agg_out, ws_out,
          src_v, dst_v, dstq_v, asg_v, adg_v, w_v, idxq_v, rows_a,
          zrows_v, zw_v, agg_sp, wsum_sp, gs_a, gs_b):
        c = lax.axis_index("c")
        s = lax.axis_index("s")
        eb = s * EC1
        lanes = lax.iota(jnp.int32, 16)
        pltpu.sync_copy(src_hbm.at[pl.ds(eb, EC1)], src_v)
        pltpu.sync_copy(dst_hbm.at[pl.ds(eb, EC1)], dst_v)
        pltpu.async_copy(as_hbm.at[src_v], asg_v, gs_a).wait()
        pltpu.async_copy(ad_hbm.at[dst_v], adg_v, gs_b).wait()

        _edge_weights(asg_v, adg_v, w_v, eb, EC1 // 16, lanes)

        def db(j, carry):
            dstq_v[j // 8, pl.ds((j % 8) * 16, 16)] = dst_v[pl.ds(j * 16, 16)]
            return carry

        lax.fori_loop(0, EC1 // 16, db, 0)
        _zero_fill(zrows_v, 16 * FW // 16)

        @pl.when(s == 0)
        def _():
            _zero_fill(zw_v, 125)

            def zw2(kk, carry):
                pltpu.sync_copy(zw_v, wsum_sp.at[pl.ds(kk * 2000, 2000)])
                return carry

            lax.fori_loop(0, 5, zw2, 0)

        def wsb(q, carry):
            pltpu.sync_copy(w_v.at[pl.ds(q * CHUNK, CHUNK)],
                            wsum_sp.at[dstq_v.at[q]], add=True)
            return carry

        for kq in range(spc):
            slab = c * spc + kq

            def ib(j, carry):
                o = j * 16
                idxq_v[j // 8, pl.ds((j % 8) * 16, 16)] = (
                    nslab * src_v[pl.ds(o, 16)] + slab)
                return carry

            lax.fori_loop(0, EC1 // 16, ib, 0)
            _zero_agg(zrows_v, agg_sp, s)
            plsc.subcore_barrier()

            def scale(rows, qbase):
                def grp(g, carry):
                    wg = w_v[pl.ds(qbase + g * 16, 16)]
                    for e16 in range(16):
                        wsp = jnp.full((16,), wg[e16])
                        row = g * 16 + e16
                        for f in range(FW // 16):
                            rows[row, pl.ds(f * 16, 16)] = (
                                rows[row, pl.ds(f * 16, 16)] * wsp)
                    return carry

                lax.fori_loop(0, CHUNK // 16, grp, 0)

            def rb(q, carry):
                g0 = pltpu.async_copy(h_hbm.at[idxq_v.at[q]], rows_a, gs_a)
                if kq == 0:
                    wsb(q, 0)
                g0.wait()
                scale(rows_a, q * CHUNK)
                pltpu.sync_copy(rows_a, agg_sp.at[dstq_v.at[q]], add=True)
                return carry

            lax.fori_loop(0, CQ1, rb, 0)
            plsc.subcore_barrier()

            pltpu.sync_copy(agg_sp.at[pl.ds(s * NPT, NPT)],
                            agg_out.at[pl.ds(slab * N + s * NPT, NPT)])

            @pl.when(s == NS - 1)
            def _():
                pltpu.sync_copy(agg_sp.at[pl.ds(NS * NPT, 16)],
                                agg_out.at[pl.ds(slab * N + NS * NPT, 16)])

        @pl.when(s == 0)
        def _():
            pltpu.sync_copy(wsum_sp, ws_out.at[c])

    return k(h_slabs, asrc, adst, src1d, dst1d)


def _sc_ugather(table, idx):
    bpw = B // (NC * NS)

    @functools.partial(
        pl.kernel,
        out_type=jax.ShapeDtypeStruct((B, DIM), jnp.float32),
        mesh=_mesh(),
        scratch_types=[
            pltpu.VMEM((bpw,), jnp.int32),
            pltpu.VMEM((bpw, DIM), jnp.float32),
            pltpu.SemaphoreType.DMA,
        ],
    )
    def k(table_hbm, idx_hbm, out_hbm, idx_v, rows_v, sem):
        wid = lax.axis_index("s") * NC + lax.axis_index("c")
        base = wid * bpw
        pltpu.sync_copy(idx_hbm.at[pl.ds(base, bpw)], idx_v)
        pltpu.async_copy(table_hbm.at[idx_v], rows_v, sem).wait()
        pltpu.sync_copy(rows_v, out_hbm.at[pl.ds(base, bpw)])

    return k(table, idx)


def kernel(u, i, graph_x, edge_index, user_emb, item_emb, W1, a1_src, a1_dst,
           b1, W2, a2_src, a2_dst, b2, c_in, c_out, r, t):
    src = edge_index[0].astype(jnp.int32)
    dst = edge_index[1].astype(jnp.int32)
    srcp = jnp.pad(src, (0, EP - E))
    dstp = jnp.pad(dst, (0, EP - E))
    x0 = jnp.take(item_emb, graph_x, axis=0)

    h1, al1s, al1d = _tc1(x0, W1, b1, a1_src, a1_dst, c_in)
    agg1, ws1 = _sc_edge(h1.reshape(4 * N, FW), al1s.reshape(N),
                         al1d.reshape(N), srcp, dstp, 4)
    h2, al2s, al2d = _tc2(agg1, ws1[0], W2, b2, a2_src, a2_dst, c_in, c_out)
    agg2, ws2 = _sc_edge(h2.reshape(2 * N, FW), al2s.reshape(N),
                         al2d.reshape(N), srcp, dstp, 2)
    u_emb = _sc_ugather(user_emb, u.astype(jnp.int32))
    h2f = _tc3a(agg2, ws2[0].reshape(N, 1), c_out)
    out = _tc3b(i, h2f, u_emb, c_out, r, t)
    return out[:, 0]


# speedup vs baseline: 1.1116x; 1.1116x over previous
import functools

import jax
import jax.numpy as jnp
from jax import lax
from jax.experimental import pallas as pl
from jax.experimental.pallas import tpu as pltpu
from jax.experimental.pallas import tpu_sc as plsc

B = 1024
DIM = 128
H1 = 256
H2 = 256
N = 10000
E = 160000
EPS = 1e-15

NC = 2
NS = 16
CHUNK = 128
EP = 163840
EC1 = EP // NS
CQ1 = EC1 // CHUNK
EC2 = EP // (NC * NS)
CQ2 = EC2 // CHUNK
NPT = 624

_INTERP = False


def _artanh(z):
    z = jnp.clip(z, -1.0 + 1e-7, 1.0 - 1e-7)
    return 0.5 * jnp.log((1.0 + z) / (1.0 - z))


def _rownorm(sq):
    return jnp.clip(jnp.sqrt(sq), EPS, None)


def _tc1_body(x_ref, w_ref, b_ref, asrc_ref, adst_ref, c_ref, h_ref, als_ref, ald_ref):
    x = x_ref[...]
    c = c_ref[0]
    sc = jnp.sqrt(c)
    n = _rownorm(jnp.sum(x * x, axis=1, keepdims=True))
    maxn = (1.0 - 1e-5) / sc
    p = jnp.where(n > maxn, x / n * maxn, x)
    pn = _rownorm(jnp.sum(p * p, axis=1, keepdims=True))
    xt = _artanh(sc * pn) * p / (sc * pn)
    h = jnp.dot(xt, w_ref[...], preferred_element_type=jnp.float32) + b_ref[...]
    h_ref[...] = h
    als_ref[...] = jnp.sum(h * asrc_ref[...], axis=1, keepdims=True)
    ald_ref[...] = jnp.sum(h * adst_ref[...], axis=1, keepdims=True)


def _tc1(x, W1, b1, a1s, a1d, c_in):
    RB = 400
    grid = (N // RB,)
    return pl.pallas_call(
        _tc1_body,
        grid=grid,
        in_specs=[
            pl.BlockSpec((RB, H1), lambda i: (i, 0)),
            pl.BlockSpec((H1, H2), lambda i: (0, 0)),
            pl.BlockSpec((1, H2), lambda i: (0, 0)),
            pl.BlockSpec((1, H2), lambda i: (0, 0)),
            pl.BlockSpec((1, H2), lambda i: (0, 0)),
            pl.BlockSpec(memory_space=pltpu.SMEM),
        ],
        out_specs=[
            pl.BlockSpec((RB, H2), lambda i: (i, 0)),
            pl.BlockSpec((RB, 1), lambda i: (i, 0)),
            pl.BlockSpec((RB, 1), lambda i: (i, 0)),
        ],
        out_shape=[
            jax.ShapeDtypeStruct((N, H2), jnp.float32),
            jax.ShapeDtypeStruct((N, 1), jnp.float32),
            jax.ShapeDtypeStruct((N, 1), jnp.float32),
        ],
        interpret=_INTERP,
    )(x, W1, b1.reshape(1, H2), a1s.reshape(1, H2), a1d.reshape(1, H2), c_in)


def _tc2_body(q0_ref, q1_ref, q2_ref, q3_ref, ws_ref, w2_ref, b2_ref,
              asrc_ref, adst_ref, cin_ref, cout_ref, h_ref, als_ref, ald_ref):
    w = ws_ref[...] + 1e-16
    qs = [jnp.maximum(q_ref[...] / w, 0.0)
          for q_ref in (q0_ref, q1_ref, q2_ref, q3_ref)]
    co = cout_ref[0]
    sco = jnp.sqrt(co)
    sq = sum(jnp.sum(q * q, axis=1, keepdims=True) for q in qs)
    un = _rownorm(sq)
    gs = jnp.tanh(sco * un) / (sco * un)
    gn = _rownorm(sq * gs * gs)
    maxn_o = (1.0 - 1e-5) / sco
    scale = gs * jnp.where(gn > maxn_o, maxn_o / gn, 1.0)
    gsq = sq * scale * scale
    ci = cin_ref[0]
    sci = jnp.sqrt(ci)
    n = _rownorm(gsq)
    maxn_i = (1.0 - 1e-5) / sci
    pscale = jnp.where(n > maxn_i, maxn_i / n, 1.0)
    pn = _rownorm(gsq * pscale * pscale)
    ls = scale * pscale * (_artanh(sci * pn) / (sci * pn))
    h = b2_ref[...]
    for qi, q in enumerate(qs):
        h = h + jnp.dot(q * ls, w2_ref[qi * FW:(qi + 1) * FW, :],
                        preferred_element_type=jnp.float32)
    h_ref[...] = h
    als_ref[...] = jnp.sum(h * asrc_ref[...], axis=1, keepdims=True)
    ald_ref[...] = jnp.sum(h * adst_ref[...], axis=1, keepdims=True)


def _tc2(agg_slabs, wsum, W2, b2, a2s, a2d, c_in, c_out):
    RB = 400
    grid = (N // RB,)
    nb = N // RB
    return pl.pallas_call(
        _tc2_body,
        grid=grid,
        in_specs=[
            pl.BlockSpec((RB, FW), lambda i: (i, 0)),
            pl.BlockSpec((RB, FW), lambda i: (i + nb, 0)),
            pl.BlockSpec((RB, FW), lambda i: (i + 2 * nb, 0)),
            pl.BlockSpec((RB, FW), lambda i: (i + 3 * nb, 0)),
            pl.BlockSpec((RB, 1), lambda i: (i, 0)),
            pl.BlockSpec((H2, DIM), lambda i: (0, 0)),
            pl.BlockSpec((1, DIM), lambda i: (0, 0)),
            pl.BlockSpec((1, DIM), lambda i: (0, 0)),
            pl.BlockSpec((1, DIM), lambda i: (0, 0)),
            pl.BlockSpec(memory_space=pltpu.SMEM),
            pl.BlockSpec(memory_space=pltpu.SMEM),
        ],
        out_specs=[
            pl.BlockSpec((RB, DIM), lambda i: (i, 0)),
            pl.BlockSpec((RB, 1), lambda i: (i, 0)),
            pl.BlockSpec((RB, 1), lambda i: (i, 0)),
        ],
        out_shape=[
            jax.ShapeDtypeStruct((N, DIM), jnp.float32),
            jax.ShapeDtypeStruct((N, 1), jnp.float32),
            jax.ShapeDtypeStruct((N, 1), jnp.float32),
        ],
        interpret=_INTERP,
    )(agg_slabs, agg_slabs, agg_slabs, agg_slabs, wsum.reshape(N, 1), W2,
      b2.reshape(1, DIM), a2s.reshape(1, DIM), a2d.reshape(1, DIM),
      c_in, c_out)


def _tc3a_body(p0_ref, p1_ref, w_ref, co_ref, h_ref):
    co = co_ref[0]
    sco = jnp.sqrt(co)
    w = w_ref[...] + 1e-16
    r0 = jnp.maximum(p0_ref[...] / w, 0.0)
    r1 = jnp.maximum(p1_ref[...] / w, 0.0)
    sq = (jnp.sum(r0 * r0, axis=1, keepdims=True)
          + jnp.sum(r1 * r1, axis=1, keepdims=True))
    un = _rownorm(sq)
    gs = jnp.tanh(sco * un) / (sco * un)
    gn = _rownorm(sq * gs * gs)
    maxn = (1.0 - 1e-5) / sco
    scale = gs * jnp.where(gn > maxn, maxn / gn, 1.0)
    h_ref[:, 0:FW] = r0 * scale
    h_ref[:, FW:DIM] = r1 * scale


def _tc3a(p_slabs, w, c_out):
    RB = 400
    nb = N // RB
    return pl.pallas_call(
        _tc3a_body,
        grid=(N // RB,),
        in_specs=[
            pl.BlockSpec((RB, FW), lambda k: (k, 0)),
            pl.BlockSpec((RB, FW), lambda k: (k + nb, 0)),
            pl.BlockSpec((RB, 1), lambda k: (k, 0)),
            pl.BlockSpec(memory_space=pltpu.SMEM),
        ],
        out_specs=pl.BlockSpec((RB, DIM), lambda k: (k, 0)),
        out_shape=jax.ShapeDtypeStruct((N, DIM), jnp.float32),
        interpret=_INTERP,
    )(p_slabs, p_slabs, w, c_out)


def _tc3b_body(i_ref, h_ref, u_ref, co_ref, r_ref, t_ref, o_ref):
    co = co_ref[0]
    sco = jnp.sqrt(co)
    maxn = (1.0 - 1e-5) / sco
    ih = jnp.dot(i_ref[...], h_ref[...], preferred_element_type=jnp.float32)
    inorm = _rownorm(jnp.sum(ih * ih, axis=1, keepdims=True))
    ih = jnp.where(inorm > maxn, ih / inorm * maxn, ih)
    u = u_ref[...]
    un2 = _rownorm(jnp.sum(u * u, axis=1, keepdims=True))
    uh = jnp.tanh(sco * un2) / (sco * un2) * u
    uhn = _rownorm(jnp.sum(uh * uh, axis=1, keepdims=True))
    uh = jnp.where(uhn > maxn, uh / uhn * maxn, uh)
    x = -uh
    x2 = jnp.sum(x * x, axis=1, keepdims=True)
    y2 = jnp.sum(ih * ih, axis=1, keepdims=True)
    xy = jnp.sum(x * ih, axis=1, keepdims=True)
    num = (1.0 + 2.0 * co * xy + co * y2) * x + (1.0 - co * x2) * ih
    den = 1.0 + 2.0 * co * xy + (co * co) * x2 * y2
    m = num / jnp.clip(den, EPS, None)
    d = jnp.sqrt(jnp.clip(jnp.sum(m * m, axis=1, keepdims=True), 0.0, None))
    dist = 2.0 / sco * _artanh(sco * d)
    o_ref[...] = 1.0 / (jnp.exp((dist - r_ref[0]) / t_ref[0]) + 1.0)


def _tc3b(i_mat, h2, u_emb, c_out, r, t):
    MB = 256
    return pl.pallas_call(
        _tc3b_body,
        grid=(B // MB,),
        in_specs=[
            pl.BlockSpec((MB, N), lambda m: (m, 0)),
            pl.BlockSpec((N, DIM), lambda m: (0, 0)),
            pl.BlockSpec((MB, DIM), lambda m: (m, 0)),
            pl.BlockSpec(memory_space=pltpu.SMEM),
            pl.BlockSpec(memory_space=pltpu.SMEM),
            pl.BlockSpec(memory_space=pltpu.SMEM),
        ],
        out_specs=pl.BlockSpec((MB, 1), lambda m: (m, 0)),
        out_shape=jax.ShapeDtypeStruct((B, 1), jnp.float32),
        interpret=_INTERP,
    )(i_mat, h2, u_emb, c_out, r, t)


def _mesh():
    return plsc.VectorSubcoreMesh(core_axis_name="c", subcore_axis_name="s",
                                  num_cores=NC, num_subcores=NS)

FW = 64


def _zero_fill(buf, nvec):
    zv = jnp.zeros((16,), jnp.float32)
    rows = buf.shape[-1] // 16 if len(buf.shape) == 2 else 0

    def zb(j, carry):
        if len(buf.shape) == 2:
            buf[j // rows, pl.ds((j % rows) * 16, 16)] = zv
        else:
            buf[pl.ds(j * 16, 16)] = zv
        return carry

    lax.fori_loop(0, nvec, zb, 0)


def _edge_weights(asg_v, adg_v, w_v, ebase, nvec, lanes):
    def wb(j, carry):
        o = j * 16
        av = asg_v[pl.ds(o, 16)] + adg_v[pl.ds(o, 16)]
        e = jnp.where(av > 0, av, 0.2 * av)
        w = jnp.exp(e)
        gid = ebase + o + lanes
        w_v[pl.ds(o, 16)] = jnp.where(gid < E, w, 0.0)
        return carry

    lax.fori_loop(0, nvec, wb, 0)


def _zero_agg(zrows_v, agg_sp, s):
    def zc(kk, carry):
        pltpu.sync_copy(zrows_v, agg_sp.at[pl.ds(s * NPT + kk * 16, 16)])
        return carry

    lax.fori_loop(0, NPT // 16, zc, 0)

    @pl.when(s == NS - 1)
    def _():
        pltpu.sync_copy(zrows_v, agg_sp.at[pl.ds(NS * NPT, 16)])


def _sc_edge(h_slabs, asrc, adst, src1d, dst1d, nslab):
    spc = nslab // NC

    @functools.partial(
        pl.kernel,
        out_type=[jax.ShapeDtypeStruct((nslab * N, FW), jnp.float32),
                  jax.ShapeDtypeStruct((NC, N), jnp.float32)],
        mesh=_mesh(),
        compiler_params=pltpu.CompilerParams(use_tc_tiling_on_sc=False),
        scratch_types=[
            pltpu.VMEM((EC1,), jnp.int32),
            pltpu.VMEM((EC1,), jnp.int32),
            pltpu.VMEM((CQ1, CHUNK), jnp.int32),
            pltpu.VMEM((EC1,), jnp.float32),
            pltpu.VMEM((EC1,), jnp.float32),
            pltpu.VMEM((EC1 + 16,), jnp.float32),
            pltpu.VMEM((CQ1, CHUNK), jnp.int32),
            pltpu.VMEM((CHUNK, FW), jnp.float32),
            pltpu.VMEM((16, FW), jnp.float32),
            pltpu.VMEM((2000,), jnp.float32),
            pltpu.VMEM_SHARED((N, FW), jnp.float32),
            pltpu.VMEM_SHARED((N,), jnp.float32),
            pltpu.SemaphoreType.DMA,
            pltpu.SemaphoreType.DMA,
        ],
    )
    def k(h_hbm, as_hbm, ad_hbm, src_hbm, dst_hbm, agg_out, ws_out,
          src_v, dst_v, dstq_v, asg_v, adg_v, w_v, idxq_v, rows_a,
          zrows_v, zw_v, agg_sp, wsum_sp, gs_a, gs_b):
        c = lax.axis_index("c")
        s = lax.axis_index("s")
        eb = s * EC1
        lanes = lax.iota(jnp.int32, 16)
        pltpu.sync_copy(src_hbm.at[pl.ds(eb, EC1)], src_v)
        pltpu.sync_copy(dst_hbm.at[pl.ds(eb, EC1)], dst_v)
        pltpu.async_copy(as_hbm.at[src_v], asg_v, gs_a).wait()
        pltpu.async_copy(ad_hbm.at[dst_v], adg_v, gs_b).wait()

        _edge_weights(asg_v, adg_v, w_v, eb, EC1 // 16, lanes)

        def db(j, carry):
            dstq_v[j // 8, pl.ds((j % 8) * 16, 16)] = dst_v[pl.ds(j * 16, 16)]
            return carry

        lax.fori_loop(0, EC1 // 16, db, 0)
        _zero_fill(zrows_v, 16 * FW // 16)

        @pl.when(s == 0)
        def _():
            _zero_fill(zw_v, 125)

            def zw2(kk, carry):
                pltpu.sync_copy(zw_v, wsum_sp.at[pl.ds(kk * 2000, 2000)])
                return carry

            lax.fori_loop(0, 5, zw2, 0)

        def wsb(q, carry):
            pltpu.sync_copy(w_v.at[pl.ds(q * CHUNK, CHUNK)],
                            wsum_sp.at[dstq_v.at[q]], add=True)
            return carry

        for kq in range(spc):
            slab = c * spc + kq

            def ib(j, carry):
                o = j * 16
                idxq_v[j // 8, pl.ds((j % 8) * 16, 16)] = (
                    nslab * src_v[pl.ds(o, 16)] + slab)
                return carry

            lax.fori_loop(0, EC1 // 16, ib, 0)
            _zero_agg(zrows_v, agg_sp, s)
            plsc.subcore_barrier()

            def scale(rows, qbase):
                def erow(e2, carry2):
                    wsp = jnp.full((16,), w_v[pl.ds(qbase + e2, 16)][0])
                    for f in range(FW // 16):
                        rows[e2, pl.ds(f * 16, 16)] = (
                            rows[e2, pl.ds(f * 16, 16)] * wsp)
                    return carry2

                lax.fori_loop(0, CHUNK, erow, 0)

            def rb(q, carry):
                g0 = pltpu.async_copy(h_hbm.at[idxq_v.at[q]], rows_a, gs_a)
                if kq == 0:
                    wsb(q, 0)
                g0.wait()
                scale(rows_a, q * CHUNK)
                pltpu.sync_copy(rows_a, agg_sp.at[dstq_v.at[q]], add=True)
                return carry

            lax.fori_loop(0, CQ1, rb, 0)
            plsc.subcore_barrier()

            pltpu.sync_copy(agg_sp.at[pl.ds(s * NPT, NPT)],
                            agg_out.at[pl.ds(slab * N + s * NPT, NPT)])

            @pl.when(s == NS - 1)
            def _():
                pltpu.sync_copy(agg_sp.at[pl.ds(NS * NPT, 16)],
                                agg_out.at[pl.ds(slab * N + NS * NPT, 16)])

        @pl.when(s == 0)
        def _():
            pltpu.sync_copy(wsum_sp, ws_out.at[c])

    return k(h_slabs, asrc, adst, src1d, dst1d)


def _sc_ugather(table, idx):
    bpw = B // (NC * NS)

    @functools.partial(
        pl.kernel,
        out_type=jax.ShapeDtypeStruct((B, DIM), jnp.float32),
        mesh=_mesh(),
        scratch_types=[
            pltpu.VMEM((bpw,), jnp.int32),
            pltpu.VMEM((bpw, DIM), jnp.float32),
            pltpu.SemaphoreType.DMA,
        ],
    )
    def k(table_hbm, idx_hbm, out_hbm, idx_v, rows_v, sem):
        wid = lax.axis_index("s") * NC + lax.axis_index("c")
        base = wid * bpw
        pltpu.sync_copy(idx_hbm.at[pl.ds(base, bpw)], idx_v)
        pltpu.async_copy(table_hbm.at[idx_v], rows_v, sem).wait()
        pltpu.sync_copy(rows_v, out_hbm.at[pl.ds(base, bpw)])

    return k(table, idx)


def kernel(u, i, graph_x, edge_index, user_emb, item_emb, W1, a1_src, a1_dst,
           b1, W2, a2_src, a2_dst, b2, c_in, c_out, r, t):
    src = edge_index[0].astype(jnp.int32)
    dst = edge_index[1].astype(jnp.int32)
    srcp = jnp.pad(src, (0, EP - E))
    dstp = jnp.pad(dst, (0, EP - E))
    x0 = jnp.take(item_emb, graph_x, axis=0)

    h1, al1s, al1d = _tc1(x0, W1, b1, a1_src, a1_dst, c_in)
    agg1, ws1 = _sc_edge(h1.reshape(4 * N, FW), al1s.reshape(N),
                         al1d.reshape(N), srcp, dstp, 4)
    h2, al2s, al2d = _tc2(agg1, ws1[0], W2, b2, a2_src, a2_dst, c_in, c_out)
    agg2, ws2 = _sc_edge(h2.reshape(2 * N, FW), al2s.reshape(N),
                         al2d.reshape(N), srcp, dstp, 2)
    u_emb = _sc_ugather(user_emb, u.astype(jnp.int32))
    h2f = _tc3a(agg2, ws2[0].reshape(N, 1), c_out)
    out = _tc3b(i, h2f, u_emb, c_out, r, t)
    return out[:, 0]


# speedup vs baseline: 1.1366x; 1.0225x over previous
import functools

import jax
import jax.numpy as jnp
from jax import lax
from jax.experimental import pallas as pl
from jax.experimental.pallas import tpu as pltpu
from jax.experimental.pallas import tpu_sc as plsc

B = 1024
DIM = 128
H1 = 256
H2 = 256
N = 10000
E = 160000
EPS = 1e-15

NC = 2
NS = 16
CHUNK = 128
EP = 163840
EC1 = EP // NS
CQ1 = EC1 // CHUNK
EC2 = EP // (NC * NS)
CQ2 = EC2 // CHUNK
NPT = 624

_INTERP = False


def _artanh(z):
    z = jnp.clip(z, -1.0 + 1e-7, 1.0 - 1e-7)
    return 0.5 * jnp.log((1.0 + z) / (1.0 - z))


def _rownorm(sq):
    return jnp.clip(jnp.sqrt(sq), EPS, None)


def _tc1_body(x_ref, w_ref, b_ref, asrc_ref, adst_ref, c_ref, h_ref, als_ref, ald_ref):
    x = x_ref[...]
    c = c_ref[0]
    sc = jnp.sqrt(c)
    n = _rownorm(jnp.sum(x * x, axis=1, keepdims=True))
    maxn = (1.0 - 1e-5) / sc
    p = jnp.where(n > maxn, x / n * maxn, x)
    pn = _rownorm(jnp.sum(p * p, axis=1, keepdims=True))
    xt = _artanh(sc * pn) * p / (sc * pn)
    h = jnp.dot(xt, w_ref[...], preferred_element_type=jnp.float32) + b_ref[...]
    h_ref[...] = h
    als_ref[...] = jnp.sum(h * asrc_ref[...], axis=1, keepdims=True)
    ald_ref[...] = jnp.sum(h * adst_ref[...], axis=1, keepdims=True)


def _tc1(x, W1, b1, a1s, a1d, c_in):
    RB = 400
    grid = (N // RB,)
    return pl.pallas_call(
        _tc1_body,
        grid=grid,
        in_specs=[
            pl.BlockSpec((RB, H1), lambda i: (i, 0)),
            pl.BlockSpec((H1, H2), lambda i: (0, 0)),
            pl.BlockSpec((1, H2), lambda i: (0, 0)),
            pl.BlockSpec((1, H2), lambda i: (0, 0)),
            pl.BlockSpec((1, H2), lambda i: (0, 0)),
            pl.BlockSpec(memory_space=pltpu.SMEM),
        ],
        out_specs=[
            pl.BlockSpec((RB, H2), lambda i: (i, 0)),
            pl.BlockSpec((RB, 1), lambda i: (i, 0)),
            pl.BlockSpec((RB, 1), lambda i: (i, 0)),
        ],
        out_shape=[
            jax.ShapeDtypeStruct((N, H2), jnp.float32),
            jax.ShapeDtypeStruct((N, 1), jnp.float32),
            jax.ShapeDtypeStruct((N, 1), jnp.float32),
        ],
        interpret=_INTERP,
    )(x, W1, b1.reshape(1, H2), a1s.reshape(1, H2), a1d.reshape(1, H2), c_in)


def _tc2_body(q0_ref, q1_ref, q2_ref, q3_ref, ws_ref, w2_ref, b2_ref,
              asrc_ref, adst_ref, cin_ref, cout_ref, h_ref, als_ref, ald_ref):
    w = ws_ref[...] + 1e-16
    qs = [jnp.maximum(q_ref[...] / w, 0.0)
          for q_ref in (q0_ref, q1_ref, q2_ref, q3_ref)]
    co = cout_ref[0]
    sco = jnp.sqrt(co)
    sq = sum(jnp.sum(q * q, axis=1, keepdims=True) for q in qs)
    un = _rownorm(sq)
    gs = jnp.tanh(sco * un) / (sco * un)
    gn = _rownorm(sq * gs * gs)
    maxn_o = (1.0 - 1e-5) / sco
    scale = gs * jnp.where(gn > maxn_o, maxn_o / gn, 1.0)
    gsq = sq * scale * scale
    ci = cin_ref[0]
    sci = jnp.sqrt(ci)
    n = _rownorm(gsq)
    maxn_i = (1.0 - 1e-5) / sci
    pscale = jnp.where(n > maxn_i, maxn_i / n, 1.0)
    pn = _rownorm(gsq * pscale * pscale)
    ls = scale * pscale * (_artanh(sci * pn) / (sci * pn))
    h = b2_ref[...]
    for qi, q in enumerate(qs):
        h = h + jnp.dot(q * ls, w2_ref[qi * FW:(qi + 1) * FW, :],
                        preferred_element_type=jnp.float32)
    h_ref[...] = h
    als_ref[...] = jnp.sum(h * asrc_ref[...], axis=1, keepdims=True)
    ald_ref[...] = jnp.sum(h * adst_ref[...], axis=1, keepdims=True)


def _tc2(agg_slabs, wsum, W2, b2, a2s, a2d, c_in, c_out):
    RB = 400
    grid = (N // RB,)
    nb = N // RB
    return pl.pallas_call(
        _tc2_body,
        grid=grid,
        in_specs=[
            pl.BlockSpec((RB, FW), lambda i: (i, 0)),
            pl.BlockSpec((RB, FW), lambda i: (i + nb, 0)),
            pl.BlockSpec((RB, FW), lambda i: (i + 2 * nb, 0)),
            pl.BlockSpec((RB, FW), lambda i: (i + 3 * nb, 0)),
            pl.BlockSpec((RB, 1), lambda i: (i, 0)),
            pl.BlockSpec((H2, DIM), lambda i: (0, 0)),
            pl.BlockSpec((1, DIM), lambda i: (0, 0)),
            pl.BlockSpec((1, DIM), lambda i: (0, 0)),
            pl.BlockSpec((1, DIM), lambda i: (0, 0)),
            pl.BlockSpec(memory_space=pltpu.SMEM),
            pl.BlockSpec(memory_space=pltpu.SMEM),
        ],
        out_specs=[
            pl.BlockSpec((RB, DIM), lambda i: (i, 0)),
            pl.BlockSpec((RB, 1), lambda i: (i, 0)),
            pl.BlockSpec((RB, 1), lambda i: (i, 0)),
        ],
        out_shape=[
            jax.ShapeDtypeStruct((N, DIM), jnp.float32),
            jax.ShapeDtypeStruct((N, 1), jnp.float32),
            jax.ShapeDtypeStruct((N, 1), jnp.float32),
        ],
        interpret=_INTERP,
    )(agg_slabs, agg_slabs, agg_slabs, agg_slabs, wsum.reshape(N, 1), W2,
      b2.reshape(1, DIM), a2s.reshape(1, DIM), a2d.reshape(1, DIM),
      c_in, c_out)


def _tc3a_body(p0_ref, p1_ref, w_ref, co_ref, h_ref):
    co = co_ref[0]
    sco = jnp.sqrt(co)
    w = w_ref[...] + 1e-16
    r0 = jnp.maximum(p0_ref[...] / w, 0.0)
    r1 = jnp.maximum(p1_ref[...] / w, 0.0)
    sq = (jnp.sum(r0 * r0, axis=1, keepdims=True)
          + jnp.sum(r1 * r1, axis=1, keepdims=True))
    un = _rownorm(sq)
    gs = jnp.tanh(sco * un) / (sco * un)
    gn = _rownorm(sq * gs * gs)
    maxn = (1.0 - 1e-5) / sco
    scale = gs * jnp.where(gn > maxn, maxn / gn, 1.0)
    h_ref[:, 0:FW] = r0 * scale
    h_ref[:, FW:DIM] = r1 * scale


def _tc3a(p_slabs, w, c_out):
    RB = 400
    nb = N // RB
    return pl.pallas_call(
        _tc3a_body,
        grid=(N // RB,),
        in_specs=[
            pl.BlockSpec((RB, FW), lambda k: (k, 0)),
            pl.BlockSpec((RB, FW), lambda k: (k + nb, 0)),
            pl.BlockSpec((RB, 1), lambda k: (k, 0)),
            pl.BlockSpec(memory_space=pltpu.SMEM),
        ],
        out_specs=pl.BlockSpec((RB, DIM), lambda k: (k, 0)),
        out_shape=jax.ShapeDtypeStruct((N, DIM), jnp.float32),
        interpret=_INTERP,
    )(p_slabs, p_slabs, w, c_out)


def _tc3b_body(i_ref, h_ref, u_ref, co_ref, r_ref, t_ref, o_ref):
    co = co_ref[0]
    sco = jnp.sqrt(co)
    maxn = (1.0 - 1e-5) / sco
    ih = jnp.dot(i_ref[...], h_ref[...], preferred_element_type=jnp.float32)
    inorm = _rownorm(jnp.sum(ih * ih, axis=1, keepdims=True))
    ih = jnp.where(inorm > maxn, ih / inorm * maxn, ih)
    u = u_ref[...]
    un2 = _rownorm(jnp.sum(u * u, axis=1, keepdims=True))
    uh = jnp.tanh(sco * un2) / (sco * un2) * u
    uhn = _rownorm(jnp.sum(uh * uh, axis=1, keepdims=True))
    uh = jnp.where(uhn > maxn, uh / uhn * maxn, uh)
    x = -uh
    x2 = jnp.sum(x * x, axis=1, keepdims=True)
    y2 = jnp.sum(ih * ih, axis=1, keepdims=True)
    xy = jnp.sum(x * ih, axis=1, keepdims=True)
    num = (1.0 + 2.0 * co * xy + co * y2) * x + (1.0 - co * x2) * ih
    den = 1.0 + 2.0 * co * xy + (co * co) * x2 * y2
    m = num / jnp.clip(den, EPS, None)
    d = jnp.sqrt(jnp.clip(jnp.sum(m * m, axis=1, keepdims=True), 0.0, None))
    dist = 2.0 / sco * _artanh(sco * d)
    o_ref[...] = 1.0 / (jnp.exp((dist - r_ref[0]) / t_ref[0]) + 1.0)


def _tc3b(i_mat, h2, u_emb, c_out, r, t):
    MB = 256
    return pl.pallas_call(
        _tc3b_body,
        grid=(B // MB,),
        in_specs=[
            pl.BlockSpec((MB, N), lambda m: (m, 0)),
            pl.BlockSpec((N, DIM), lambda m: (0, 0)),
            pl.BlockSpec((MB, DIM), lambda m: (m, 0)),
            pl.BlockSpec(memory_space=pltpu.SMEM),
            pl.BlockSpec(memory_space=pltpu.SMEM),
            pl.BlockSpec(memory_space=pltpu.SMEM),
        ],
        out_specs=pl.BlockSpec((MB, 1), lambda m: (m, 0)),
        out_shape=jax.ShapeDtypeStruct((B, 1), jnp.float32),
        interpret=_INTERP,
    )(i_mat, h2, u_emb, c_out, r, t)


def _mesh():
    return plsc.VectorSubcoreMesh(core_axis_name="c", subcore_axis_name="s",
                                  num_cores=NC, num_subcores=NS)

FW = 64


def _zero_fill(buf, nvec):
    zv = jnp.zeros((16,), jnp.float32)
    rows = buf.shape[-1] // 16 if len(buf.shape) == 2 else 0

    def zb(j, carry):
        if len(buf.shape) == 2:
            buf[j // rows, pl.ds((j % rows) * 16, 16)] = zv
        else:
            buf[pl.ds(j * 16, 16)] = zv
        return carry

    lax.fori_loop(0, nvec, zb, 0)


def _edge_weights(asg_v, adg_v, w_v, ebase, nvec, lanes):
    def wb(j, carry):
        o = j * 16
        av = asg_v[pl.ds(o, 16)] + adg_v[pl.ds(o, 16)]
        e = jnp.where(av > 0, av, 0.2 * av)
        w = jnp.exp(e)
        gid = ebase + o + lanes
        w_v[pl.ds(o, 16)] = jnp.where(gid < E, w, 0.0)
        return carry

    lax.fori_loop(0, nvec, wb, 0)


def _zero_agg(zrows_v, agg_sp, s):
    def zc(kk, carry):
        pltpu.sync_copy(zrows_v, agg_sp.at[pl.ds(s * NPT + kk * 16, 16)])
        return carry

    lax.fori_loop(0, NPT // 16, zc, 0)

    @pl.when(s == NS - 1)
    def _():
        pltpu.sync_copy(zrows_v, agg_sp.at[pl.ds(NS * NPT, 16)])


def _sc_edge(h_slabs, asrc, adst, src1d, dst1d, nslab):
    spc = nslab // NC

    @functools.partial(
        pl.kernel,
        out_type=[jax.ShapeDtypeStruct((nslab * N, FW), jnp.float32),
                  jax.ShapeDtypeStruct((NC, N), jnp.float32)],
        mesh=_mesh(),
        compiler_params=pltpu.CompilerParams(use_tc_tiling_on_sc=False),
        scratch_types=[
            pltpu.VMEM((EC1,), jnp.int32),
            pltpu.VMEM((EC1,), jnp.int32),
            pltpu.VMEM((CQ1, CHUNK), jnp.int32),
            pltpu.VMEM((EC1,), jnp.float32),
            pltpu.VMEM((EC1,), jnp.float32),
            pltpu.VMEM((EC1 + 16,), jnp.float32),
            pltpu.VMEM((CQ1, CHUNK), jnp.int32),
            pltpu.VMEM((CHUNK, FW), jnp.float32),
            pltpu.VMEM((16, FW), jnp.float32),
            pltpu.VMEM((2000,), jnp.float32),
            pltpu.VMEM_SHARED((N, FW), jnp.float32),
            pltpu.VMEM_SHARED((N,), jnp.float32),
            pltpu.SemaphoreType.DMA,
            pltpu.SemaphoreType.DMA,
        ],
    )
    def k(h_hbm, as_hbm, ad_hbm, src_hbm, dst_hbm, agg_out, ws_out,
          src_v, dst_v, dstq_v, asg_v, adg_v, w_v, idxq_v, rows_a,
          zrows_v, zw_v, agg_sp, wsum_sp, gs_a, gs_b):
        c = lax.axis_index("c")
        s = lax.axis_index("s")
        eb = s * EC1
        lanes = lax.iota(jnp.int32, 16)
        pltpu.sync_copy(src_hbm.at[pl.ds(eb, EC1)], src_v)
        pltpu.sync_copy(dst_hbm.at[pl.ds(eb, EC1)], dst_v)
        pltpu.async_copy(as_hbm.at[src_v], asg_v, gs_a).wait()
        pltpu.async_copy(ad_hbm.at[dst_v], adg_v, gs_b).wait()

        _edge_weights(asg_v, adg_v, w_v, eb, EC1 // 16, lanes)

        def db(j, carry):
            dstq_v[j // 8, pl.ds((j % 8) * 16, 16)] = dst_v[pl.ds(j * 16, 16)]
            return carry

        lax.fori_loop(0, EC1 // 16, db, 0)
        _zero_fill(zrows_v, 16 * FW // 16)

        @pl.when(s == 0)
        def _():
            _zero_fill(zw_v, 125)

            def zw2(kk, carry):
                pltpu.sync_copy(zw_v, wsum_sp.at[pl.ds(kk * 2000, 2000)])
                return carry

            lax.fori_loop(0, 5, zw2, 0)

        def wsb(q, carry):
            pltpu.sync_copy(w_v.at[pl.ds(q * CHUNK, CHUNK)],
                            wsum_sp.at[dstq_v.at[q]], add=True)
            return carry

        for kq in range(spc):
            slab = c * spc + kq

            def ib(j, carry):
                o = j * 16
                idxq_v[j // 8, pl.ds((j % 8) * 16, 16)] = (
                    nslab * src_v[pl.ds(o, 16)] + slab)
                return carry

            lax.fori_loop(0, EC1 // 16, ib, 0)
            _zero_agg(zrows_v, agg_sp, s)
            plsc.subcore_barrier()

            def scale(rows, qbase):
                def erow(e4, carry2):
                    for u in range(4):
                        e2 = e4 * 4 + u
                        wsp = jnp.full((16,), w_v[pl.ds(qbase + e2, 16)][0])
                        for f in range(FW // 16):
                            rows[e2, pl.ds(f * 16, 16)] = (
                                rows[e2, pl.ds(f * 16, 16)] * wsp)
                    return carry2

                lax.fori_loop(0, CHUNK // 4, erow, 0)

            def rb(q, carry):
                g0 = pltpu.async_copy(h_hbm.at[idxq_v.at[q]], rows_a, gs_a)
                if kq == 0:
                    wsb(q, 0)
                g0.wait()
                scale(rows_a, q * CHUNK)
                pltpu.sync_copy(rows_a, agg_sp.at[dstq_v.at[q]], add=True)
                return carry

            lax.fori_loop(0, CQ1, rb, 0)
            plsc.subcore_barrier()

            pltpu.sync_copy(agg_sp.at[pl.ds(s * NPT, NPT)],
                            agg_out.at[pl.ds(slab * N + s * NPT, NPT)])

            @pl.when(s == NS - 1)
            def _():
                pltpu.sync_copy(agg_sp.at[pl.ds(NS * NPT, 16)],
                                agg_out.at[pl.ds(slab * N + NS * NPT, 16)])

        @pl.when(s == 0)
        def _():
            pltpu.sync_copy(wsum_sp, ws_out.at[c])

    return k(h_slabs, asrc, adst, src1d, dst1d)


def _sc_ugather(table, idx):
    bpw = B // (NC * NS)

    @functools.partial(
        pl.kernel,
        out_type=jax.ShapeDtypeStruct((B, DIM), jnp.float32),
        mesh=_mesh(),
        scratch_types=[
            pltpu.VMEM((bpw,), jnp.int32),
            pltpu.VMEM((bpw, DIM), jnp.float32),
            pltpu.SemaphoreType.DMA,
        ],
    )
    def k(table_hbm, idx_hbm, out_hbm, idx_v, rows_v, sem):
        wid = lax.axis_index("s") * NC + lax.axis_index("c")
        base = wid * bpw
        pltpu.sync_copy(idx_hbm.at[pl.ds(base, bpw)], idx_v)
        pltpu.async_copy(table_hbm.at[idx_v], rows_v, sem).wait()
        pltpu.sync_copy(rows_v, out_hbm.at[pl.ds(base, bpw)])

    return k(table, idx)


def kernel(u, i, graph_x, edge_index, user_emb, item_emb, W1, a1_src, a1_dst,
           b1, W2, a2_src, a2_dst, b2, c_in, c_out, r, t):
    src = edge_index[0].astype(jnp.int32)
    dst = edge_index[1].astype(jnp.int32)
    srcp = jnp.pad(src, (0, EP - E))
    dstp = jnp.pad(dst, (0, EP - E))
    x0 = jnp.take(item_emb, graph_x, axis=0)

    h1, al1s, al1d = _tc1(x0, W1, b1, a1_src, a1_dst, c_in)
    agg1, ws1 = _sc_edge(h1.reshape(4 * N, FW), al1s.reshape(N),
                         al1d.reshape(N), srcp, dstp, 4)
    h2, al2s, al2d = _tc2(agg1, ws1[0], W2, b2, a2_src, a2_dst, c_in, c_out)
    agg2, ws2 = _sc_edge(h2.reshape(2 * N, FW), al2s.reshape(N),
                         al2d.reshape(N), srcp, dstp, 2)
    u_emb = _sc_ugather(user_emb, u.astype(jnp.int32))
    h2f = _tc3a(agg2, ws2[0].reshape(N, 1), c_out)
    out = _tc3b(i, h2f, u_emb, c_out, r, t)
    return out[:, 0]
